# Initial kernel scaffold; baseline (speedup 1.0000x reference)
import jax
import jax.numpy as jnp
from jax.experimental import pallas as pl


def kernel(x, w0, b0, w1, b1, w2, b2, w3, b3, w4, b4, wf0, bf0, wf1, bf1, wf2, bf2, wf3, bf3):
    raise NotImplementedError("write your pallas kernel here")


# speedup vs baseline: 1.0179x; 1.0179x over previous
import jax
import jax.numpy as jnp
from jax.experimental import pallas as pl
from jax.experimental.pallas import tpu as pltpu

_OH2, _OW2 = 5, 22
_OH3, _OW3 = 3, 20
_OW4 = 18
_TB = 8


def _elu(x):
    return jnp.where(x > 0, x, jnp.exp(jnp.minimum(x, 0.0)) - 1.0)


def _mm_elu_kernel(p_ref, w_ref, b_ref, o_ref):
    acc = jnp.dot(p_ref[...], w_ref[...], preferred_element_type=jnp.float32)
    o_ref[...] = _elu(acc + b_ref[...]).astype(o_ref.dtype)


def _mm_elu(patches, w, b, tm):
    M, K = patches.shape
    Cout = w.shape[1]
    return pl.pallas_call(
        _mm_elu_kernel,
        out_shape=jax.ShapeDtypeStruct((M, Cout), jnp.bfloat16),
        grid=(pl.cdiv(M, tm),),
        in_specs=[
            pl.BlockSpec((tm, K), lambda i: (i, 0)),
            pl.BlockSpec((K, Cout), lambda i: (0, 0)),
            pl.BlockSpec((1, Cout), lambda i: (0, 0)),
        ],
        out_specs=pl.BlockSpec((tm, Cout), lambda i: (i, 0)),
        compiler_params=pltpu.CompilerParams(dimension_semantics=("parallel",)),
    )(patches, w, b)


def _tail_kernel(p2_ref, w2_ref, b2_ref, w3_ref, b3_ref, w4_ref, b4_ref,
                 wf0_ref, bf0_ref, wf1_ref, bf1_ref, wf2_ref, bf2_ref,
                 wf3_ref, bf3_ref, o_ref, f_s):
    B = _TB
    p2 = p2_ref[...].reshape(B * _OH2 * _OW2, 900)
    acc2 = jnp.dot(p2, w2_ref[...], preferred_element_type=jnp.float32)
    a2 = _elu(acc2 + b2_ref[...]).astype(jnp.bfloat16)
    a2 = a2.reshape(B, _OH2 * _OW2, 48)

    acc3 = jnp.zeros((B * 64, 64), jnp.float32)
    for i in range(3):
        for j in range(3):
            blk = a2[:, i * 22 + j: i * 22 + j + 64, :].reshape(B * 64, 48)
            acc3 = acc3 + jnp.dot(blk, w3_ref[i * 3 + j],
                                  preferred_element_type=jnp.float32)
    a3 = _elu(acc3 + b3_ref[...]).astype(jnp.bfloat16).reshape(B, 64, 64)

    p4 = jnp.concatenate(
        [a3[:, (i * 22 + j): (i * 22 + j) + _OW4, :]
         for i in range(3) for j in range(3)], axis=-1)
    acc4 = jnp.dot(p4.reshape(B * _OW4, 9 * 64), w4_ref[...],
                   preferred_element_type=jnp.float32)
    a4 = _elu(acc4 + b4_ref[...]).reshape(B, _OW4, 64)

    for w in range(_OW4):
        f_s[:, 64 * w: 64 * (w + 1)] = a4[:, w, :].astype(jnp.bfloat16)
    h = _elu(jnp.dot(f_s[...], wf0_ref[...],
                     preferred_element_type=jnp.float32) + bf0_ref[...])
    h = _elu(jnp.dot(h.astype(jnp.bfloat16), wf1_ref[...],
                     preferred_element_type=jnp.float32) + bf1_ref[...])
    h = _elu(jnp.dot(h.astype(jnp.bfloat16), wf2_ref[...],
                     preferred_element_type=jnp.float32) + bf2_ref[...])
    y = jnp.sum(h * wf3_ref[...], axis=-1, keepdims=True) + bf3_ref[...]
    o_ref[...] = jnp.broadcast_to(y, (B, 128))


def _tail(p2, weights):
    N = p2.shape[0]

    def full(a):
        return pl.BlockSpec(a.shape, lambda n: (0,) * a.ndim)

    out = pl.pallas_call(
        _tail_kernel,
        out_shape=jax.ShapeDtypeStruct((N, 128), jnp.float32),
        grid=(N // _TB,),
        in_specs=[pl.BlockSpec((_TB,) + p2.shape[1:], lambda n: (n, 0, 0))]
                 + [full(a) for a in weights],
        out_specs=pl.BlockSpec((_TB, 128), lambda n: (n, 0)),
        scratch_shapes=[pltpu.VMEM((_TB, _OW4 * 64), jnp.bfloat16)],
        compiler_params=pltpu.CompilerParams(dimension_semantics=("parallel",)),
    )(p2, *weights)
    return out[:, :1]


def _im2col(x, kh, kw, sh, sw):
    N, H, W, C = x.shape
    OH = (H - kh) // sh + 1
    OW = (W - kw) // sw + 1
    taps = []
    for i in range(kh):
        for j in range(kw):
            taps.append(x[:, i:i + sh * (OH - 1) + 1:sh,
                            j:j + sw * (OW - 1) + 1:sw, :])
    return jnp.concatenate(taps, axis=-1), OH, OW


def kernel(x, w0, b0, w1, b1, w2, b2, w3, b3, w4, b4,
           wf0, bf0, wf1, bf1, wf2, bf2, wf3, bf3):
    N = x.shape[0]
    xh = jnp.transpose(x, (0, 2, 3, 1)).astype(jnp.bfloat16)

    p0, oh0, ow0 = _im2col(xh, 5, 5, 2, 2)
    a0 = _mm_elu(p0.reshape(N * oh0 * ow0, -1), w0, b0, tm=4096)
    a0 = a0.reshape(N, oh0, ow0, -1)

    p1, oh1, ow1 = _im2col(a0, 5, 5, 2, 2)
    a1 = _mm_elu(p1.reshape(N * oh1 * ow1, -1), w1, b1, tm=4096)
    a1 = a1.reshape(N, oh1, ow1, -1)

    p2, oh2, ow2 = _im2col(a1, 5, 5, 2, 2)
    p2 = p2.reshape(N, oh2 * ow2, -1)
    tail_w = (w2, b2, w3, b3, w4, b4, wf0, bf0, wf1, bf1, wf2, bf2, wf3, bf3)
    return _tail(p2, tail_w)


# speedup vs baseline: 69.7925x; 68.5668x over previous
import jax
import jax.numpy as jnp
from jax.experimental import pallas as pl
from jax.experimental.pallas import tpu as pltpu

_B = 8

_L0 = (66, 200, 3, 31, 98, 24, 5, 2)
_L1 = (31, 98, 24, 14, 47, 36, 5, 2)
_L2 = (14, 47, 36, 5, 22, 48, 5, 2)
_L3 = (5, 22, 48, 3, 20, 64, 3, 1)
_L4 = (3, 20, 64, 1, 18, 64, 3, 1)


def _elu(x):
    return jnp.where(x > 0, x, jnp.exp(jnp.minimum(x, 0.0)) - 1.0)


def _band(h_in, oh, k, stride):
    h = jax.lax.broadcasted_iota(jnp.int32, (h_in, oh, k), 0)
    o = jax.lax.broadcasted_iota(jnp.int32, (h_in, oh, k), 1)
    i = jax.lax.broadcasted_iota(jnp.int32, (h_in, oh, k), 2)
    return (h == stride * o + i).astype(jnp.bfloat16)


def _banded_weights(w, geom, c_major_in=False):
    h_in, _, c_in, oh, _, c_out, k, s = geom
    wr = w.reshape(k, k, c_in, c_out).astype(jnp.bfloat16)
    band = _band(h_in, oh, k, s)
    if c_major_in:
        m = jnp.einsum('hoi,ijck->jchok', band, wr)
        return m.reshape(k, c_in * h_in, oh * c_out)
    m = jnp.einsum('hoi,ijck->jhcok', band, wr)
    return m.reshape(k, h_in * c_in, oh * c_out)


def _conv_s2(in_ref, wb_ref, bias, geom, out_ref, p_out):
    _, _, _, oh, ow, c_out, k, _ = geom
    bsz, p_in, _, lanes = in_ref.shape
    for q in range(p_out):
        m_q = (ow - 1 - q) // p_out + 1
        acc = jnp.zeros((bsz * m_q, oh * c_out), jnp.float32)
        for j in range(k):
            t = 2 * q + j
            r, st = t % p_in, t // p_in
            slab = in_ref[:, r, st: st + m_q, :].reshape(bsz * m_q, lanes)
            acc = acc + jnp.dot(slab.astype(jnp.bfloat16), wb_ref[j],
                                preferred_element_type=jnp.float32)
        res = _elu(acc + bias).reshape(bsz, m_q, oh * c_out)
        if p_out == 1:
            out_ref[:, :m_q, :] = res
        else:
            out_ref[:, q, :m_q, :] = res


def _conv_s1(in_ref, wb_ref, bias, geom, out_ref):
    _, _, _, oh, ow, c_out, k, _ = geom
    bsz = in_ref.shape[0]
    lanes = in_ref.shape[2]
    acc = jnp.zeros((bsz * ow, oh * c_out), jnp.float32)
    for j in range(k):
        slab = in_ref[:, j: j + ow, :].reshape(bsz * ow, lanes)
        acc = acc + jnp.dot(slab.astype(jnp.bfloat16), wb_ref[j],
                            preferred_element_type=jnp.float32)
    out_ref[...] = _elu(acc + bias).reshape(bsz, ow, oh * c_out)


def _fwd_kernel(xp_ref, w0_ref, b0_ref, w1_ref, b1_ref, w2_ref, b2_ref,
                w3_ref, b3_ref, w4_ref, b4_ref,
                wf0_ref, bf0_ref, wf1_ref, bf1_ref, wf2_ref, bf2_ref,
                wf3_ref, bf3_ref, o_ref, a0_s, a1_s, a2_s, a3_s, a4_s, f_s):
    _conv_s2(xp_ref, w0_ref, b0_ref[...], _L0, a0_s, 4)
    _conv_s2(a0_s, w1_ref, b1_ref[...], _L1, a1_s, 2)
    _conv_s2(a1_s, w2_ref, b2_ref[...], _L2, a2_s, 1)
    _conv_s1(a2_s, w3_ref, b3_ref[...], _L3, a3_s)
    _conv_s1(a3_s, w4_ref, b4_ref[...], _L4, a4_s)

    for w in range(18):
        f_s[:, 64 * w: 64 * (w + 1)] = a4_s[:, w, :].astype(jnp.bfloat16)
    h = _elu(jnp.dot(f_s[...], wf0_ref[...],
                     preferred_element_type=jnp.float32) + bf0_ref[...])
    h = _elu(jnp.dot(h.astype(jnp.bfloat16), wf1_ref[...],
                     preferred_element_type=jnp.float32) + bf1_ref[...])
    h = _elu(jnp.dot(h.astype(jnp.bfloat16), wf2_ref[...],
                     preferred_element_type=jnp.float32) + bf2_ref[...])
    y = jnp.sum(h * wf3_ref[...], axis=-1, keepdims=True) + bf3_ref[...]
    o_ref[...] = jnp.broadcast_to(y, (_B, 128))


def kernel(x, w0, b0, w1, b1, w2, b2, w3, b3, w4, b4,
           wf0, bf0, wf1, bf1, wf2, bf2, wf3, bf3):
    n = x.shape[0]
    xt = jnp.swapaxes(x.reshape(n, 198, 200), 1, 2)
    xp = jnp.transpose(xt.reshape(n, 25, 8, 198), (0, 2, 1, 3))

    wb0 = _banded_weights(w0, _L0, c_major_in=True)
    wb1 = _banded_weights(w1, _L1)
    wb2 = _banded_weights(w2, _L2)
    wb3 = _banded_weights(w3.reshape(9 * 48, 64), _L3)
    wb4 = _banded_weights(w4, _L4)

    tb0 = jnp.tile(b0, (1, _L0[3]))
    tb1 = jnp.tile(b1, (1, _L1[3]))
    tb2 = jnp.tile(b2, (1, _L2[3]))
    tb3 = jnp.tile(b3, (1, _L3[3]))
    tb4 = jnp.tile(b4, (1, _L4[3]))

    def full(a):
        return pl.BlockSpec(a.shape, lambda i: (0,) * a.ndim)

    weights = (wb0, tb0, wb1, tb1, wb2, tb2, wb3, tb3, wb4, tb4,
               wf0, bf0, wf1, bf1, wf2, bf2, wf3, bf3)
    out = pl.pallas_call(
        _fwd_kernel,
        out_shape=jax.ShapeDtypeStruct((n, 128), jnp.float32),
        grid=(n // _B,),
        in_specs=[pl.BlockSpec((_B, 8, 25, 198), lambda i: (i, 0, 0, 0))]
                 + [full(a) for a in weights],
        out_specs=pl.BlockSpec((_B, 128), lambda i: (i, 0)),
        scratch_shapes=[
            pltpu.VMEM((_B, 4, 25, 744), jnp.float32),
            pltpu.VMEM((_B, 2, 24, 504), jnp.float32),
            pltpu.VMEM((_B, 22, 240), jnp.float32),
            pltpu.VMEM((_B, 20, 192), jnp.float32),
            pltpu.VMEM((_B, 18, 64), jnp.float32),
            pltpu.VMEM((_B, 18 * 64), jnp.bfloat16),
        ],
        compiler_params=pltpu.CompilerParams(dimension_semantics=("parallel",)),
    )(xp, *weights)
    return out[:, :1]
